# baseline (speedup 1.0000x reference)
import functools

import jax
import jax.numpy as jnp
from jax import lax
from jax.experimental import pallas as pl
from jax.experimental.pallas import tpu as pltpu
from jax.experimental.pallas import tpu_sc as plsc

N = 10000
E = 320000
D_IN = 128
D_HID = 256
D_OUT = 128

NC = 2
NS = 16
CH = 128
SUP = 8
NPAD = 10240
ER = 2560
E_PAD = ER * CH
DUMMY_DST = 10016

_mesh = plsc.VectorSubcoreMesh(core_axis_name="c", subcore_axis_name="s")


NSUP = ER // SUP


def _degree_body(dst2_hbm, out_hbm, didx_v, ones_v, zb_v, ssem, acc_sh):
  c = lax.axis_index("c")
  s = lax.axis_index("s")
  w = s * NC + c

  @pl.loop(0, CH // 16)
  def _(i):
    ones_v[pl.ds(i * 16, 16)] = jnp.ones((16,), jnp.float32)

  @pl.loop(0, 640 // 16)
  def _(i):
    zb_v[pl.ds(i * 16, 16)] = jnp.zeros((16,), jnp.float32)

  pltpu.sync_copy(zb_v, acc_sh.at[pl.ds(s * 640, 640)])
  plsc.subcore_barrier()

  cnt = NSUP // (NC * NS)

  @pl.loop(0, cnt)
  def _(j):
    r0 = (w * cnt + j) * SUP
    pltpu.sync_copy(dst2_hbm.at[pl.ds(r0, SUP)], didx_v)
    for k in range(SUP):
      pltpu.async_copy(ones_v, acc_sh.at[didx_v.at[k]], ssem, add=True)
    for k in range(SUP):
      pltpu.make_async_copy(ones_v, acc_sh.at[didx_v.at[k]], ssem).wait()

  plsc.subcore_barrier()
  pltpu.sync_copy(acc_sh.at[pl.ds(s * 640, 640)],
                  out_hbm.at[c].at[pl.ds(s * 640, 640)])


_sc_degree = pl.kernel(
    _degree_body,
    out_type=jax.ShapeDtypeStruct((NC, NPAD), jnp.float32),
    mesh=_mesh,
    scratch_types=[
        pltpu.VMEM((SUP, CH), jnp.int32),
        pltpu.VMEM((CH,), jnp.float32),
        pltpu.VMEM((640,), jnp.float32),
        pltpu.SemaphoreType.DMA,
        pltpu.VMEM_SHARED((NPAD,), jnp.float32),
    ],
)


def _scatter_body(split_edges, src2_hbm, dst2_hbm, tbl_hbm, out_hbm,
                  sidx0_v, didx0_v, sidx1_v, didx1_v,
                  rows0_v, rows1_v,
                  gsem0, gsem1, isem0, isem1, ssem, acc_sh):
  c = lax.axis_index("c")
  s = lax.axis_index("s")
  rows = (rows0_v, rows1_v)
  gsems = (gsem0, gsem1)
  sidx = (sidx0_v, sidx1_v)
  didx = (didx0_v, didx1_v)
  isems = (isem0, isem1)

  if split_edges:
    w = s * NC + c
    nw = NC * NS
  else:
    w = s
    nw = NS
  cnt = NSUP // nw

  def row0_of(j):
    return (w * cnt + j) * SUP

  def start_idx(j, ib):
    r0 = row0_of(j)
    pltpu.async_copy(src2_hbm.at[pl.ds(r0, SUP)], sidx[ib], isems[ib])
    pltpu.async_copy(dst2_hbm.at[pl.ds(r0, SUP)], didx[ib], isems[ib])

  def wait_idx(j, ib):
    r0 = row0_of(j)
    pltpu.make_async_copy(src2_hbm.at[pl.ds(r0, SUP)], sidx[ib],
                          isems[ib]).wait()
    pltpu.make_async_copy(dst2_hbm.at[pl.ds(r0, SUP)], didx[ib],
                          isems[ib]).wait()

  start_idx(0, 0)

  @pl.loop(0, CH)
  def _(r):
    @pl.loop(0, 8)
    def _(j):
      rows0_v[r, pl.ds(j * 16, 16)] = jnp.zeros((16,), jnp.float32)

  for i in range(5):
    pltpu.async_copy(rows0_v, acc_sh.at[pl.ds(s * 640 + i * 128, 128)], ssem)
  for i in range(5):
    pltpu.make_async_copy(rows0_v, acc_sh.at[pl.ds(s * 640 + i * 128, 128)],
                          ssem).wait()

  plsc.subcore_barrier()

  tbl_c = tbl_hbm if split_edges else tbl_hbm.at[c]

  def start_gather(idx_row, b):
    pltpu.async_copy(tbl_c.at[idx_row], rows[b], gsems[b])

  def wait_gather(idx_row, b):
    pltpu.make_async_copy(tbl_c.at[idx_row], rows[b], gsems[b]).wait()

  def drain_scatter(b, idx_row):
    pltpu.make_async_copy(rows[b], acc_sh.at[idx_row], ssem).wait()

  def process_super(ib):
    si = sidx[ib]
    di = didx[ib]
    start_gather(si.at[0], 0)
    for k in range(SUP):
      b = k & 1
      if k + 1 < SUP:
        if k >= 1:
          drain_scatter(1 - b, di.at[k - 1])
        start_gather(si.at[k + 1], 1 - b)
      wait_gather(si.at[k], b)
      pltpu.async_copy(rows[b], acc_sh.at[di.at[k]], ssem, add=True)
    drain_scatter(0, di.at[SUP - 2])
    drain_scatter(1, di.at[SUP - 1])

  @pl.loop(0, cnt // 2)
  def _(p):
    j0 = 2 * p
    j1 = j0 + 1
    wait_idx(j0, 0)
    start_idx(j1, 1)
    process_super(0)
    wait_idx(j1, 1)

    @pl.when(j1 + 1 < cnt)
    def _():
      start_idx(j1 + 1, 0)

    process_super(1)

  plsc.subcore_barrier()

  @pl.when(s < NS - 1)
  def _():
    pltpu.sync_copy(acc_sh.at[pl.ds(s * 640, 640)],
                    out_hbm.at[c].at[pl.ds(s * 640, 640)])

  @pl.when(s == NS - 1)
  def _():
    pltpu.sync_copy(acc_sh.at[pl.ds(s * 640, 400)],
                    out_hbm.at[c].at[pl.ds(s * 640, 400)])


def _make_scatter(split_edges):
  return pl.kernel(
      functools.partial(_scatter_body, split_edges),
      out_type=jax.ShapeDtypeStruct((NC, N, 128), jnp.float32),
      mesh=_mesh,
      scratch_types=[
          pltpu.VMEM((SUP, CH), jnp.int32),
          pltpu.VMEM((SUP, CH), jnp.int32),
          pltpu.VMEM((SUP, CH), jnp.int32),
          pltpu.VMEM((SUP, CH), jnp.int32),
          pltpu.VMEM((CH, 128), jnp.float32),
          pltpu.VMEM((CH, 128), jnp.float32),
          pltpu.SemaphoreType.DMA,
          pltpu.SemaphoreType.DMA,
          pltpu.SemaphoreType.DMA,
          pltpu.SemaphoreType.DMA,
          pltpu.SemaphoreType.DMA,
          pltpu.VMEM_SHARED((NPAD, 128), jnp.float32),
      ],
  )


_sc_scatter1 = _make_scatter(False)
_sc_scatter2 = _make_scatter(True)


R = 1000


def _mm1_body(x_ref, w_ref, o_ref):
  o_ref[...] = jnp.dot(x_ref[...], w_ref[...],
                       preferred_element_type=jnp.float32)


def _tc_matmul1(x, W1):
  return pl.pallas_call(
      _mm1_body,
      grid=(N // R,),
      in_specs=[
          pl.BlockSpec((R, D_IN), lambda i: (i, 0)),
          pl.BlockSpec((D_IN, D_HID), lambda i: (0, 0)),
      ],
      out_specs=pl.BlockSpec((R, D_HID), lambda i: (i, 0)),
      out_shape=jax.ShapeDtypeStruct((N, D_HID), jnp.float32),
  )(x, W1)


def _scale_body(degp_ref, xw_ref, dinv_ref, g_ref):
  deg = degp_ref[0] + degp_ref[1] + 1.0
  dinv = lax.rsqrt(deg)
  g = xw_ref[...] * dinv
  dinv_ref[...] = dinv
  g_ref[0] = g[:, :128]
  g_ref[1] = g[:, 128:]


def _tc_scale(degp, xw):
  return pl.pallas_call(
      _scale_body,
      grid=(N // R,),
      in_specs=[
          pl.BlockSpec((NC, R, 1), lambda i: (0, i, 0)),
          pl.BlockSpec((R, D_HID), lambda i: (i, 0)),
      ],
      out_specs=[
          pl.BlockSpec((R, 1), lambda i: (i, 0)),
          pl.BlockSpec((NC, R, 128), lambda i: (0, i, 0)),
      ],
      out_shape=[
          jax.ShapeDtypeStruct((N, 1), jnp.float32),
          jax.ShapeDtypeStruct((NC, N, 128), jnp.float32),
      ],
  )(degp, xw)


def _mid_body(a_ref, g1_ref, dinv_ref, b1_ref, w2_ref, g2_ref):
  dinv = dinv_ref[...]
  h_lo = jnp.maximum(dinv * (a_ref[0] + g1_ref[0]) + b1_ref[:, :128], 0.0)
  h_hi = jnp.maximum(dinv * (a_ref[1] + g1_ref[1]) + b1_ref[:, 128:], 0.0)
  xw2 = (jnp.dot(h_lo, w2_ref[:128], preferred_element_type=jnp.float32) +
         jnp.dot(h_hi, w2_ref[128:], preferred_element_type=jnp.float32))
  g2_ref[...] = dinv * xw2


def _tc_mid(a1, g1, dinv, b1, W2):
  return pl.pallas_call(
      _mid_body,
      grid=(N // R,),
      in_specs=[
          pl.BlockSpec((NC, R, 128), lambda i: (0, i, 0)),
          pl.BlockSpec((NC, R, 128), lambda i: (0, i, 0)),
          pl.BlockSpec((R, 1), lambda i: (i, 0)),
          pl.BlockSpec((1, D_HID), lambda i: (0, 0)),
          pl.BlockSpec((D_HID, D_OUT), lambda i: (0, 0)),
      ],
      out_specs=pl.BlockSpec((R, 128), lambda i: (i, 0)),
      out_shape=jax.ShapeDtypeStruct((N, 128), jnp.float32),
  )(a1, g1, dinv, b1, W2)


def _post_body(a2_ref, g2_ref, dinv_ref, b2_ref, o_ref):
  o_ref[...] = (dinv_ref[...] * (a2_ref[0] + a2_ref[1] + g2_ref[...])
                + b2_ref[...])


def _tc_post(a2, g2, dinv, b2):
  return pl.pallas_call(
      _post_body,
      grid=(N // R,),
      in_specs=[
          pl.BlockSpec((NC, R, 128), lambda i: (0, i, 0)),
          pl.BlockSpec((R, 128), lambda i: (i, 0)),
          pl.BlockSpec((R, 1), lambda i: (i, 0)),
          pl.BlockSpec((1, D_OUT), lambda i: (0, 0)),
      ],
      out_specs=pl.BlockSpec((R, D_OUT), lambda i: (i, 0)),
      out_shape=jax.ShapeDtypeStruct((N, D_OUT), jnp.float32),
  )(a2, g2, dinv, b2)


def kernel(x, edge_index, W1, b1, W2, b2):
  npad = E_PAD - E
  src2 = jnp.concatenate(
      [edge_index[0], jnp.zeros((npad,), jnp.int32)]).reshape(ER, CH)
  dst2 = jnp.concatenate(
      [edge_index[1], jnp.full((npad,), DUMMY_DST, jnp.int32)]).reshape(ER, CH)
  xw = _tc_matmul1(x, W1)
  degp = _sc_degree(dst2)
  dinv, g1 = _tc_scale(degp.reshape(NC, NPAD, 1), xw)
  a1 = _sc_scatter1(src2, dst2, g1)
  g2 = _tc_mid(a1, g1, dinv, b1.reshape(1, D_HID), W2)
  a2 = _sc_scatter2(src2, dst2, g2)
  return _tc_post(a2, g2, dinv, b2.reshape(1, D_OUT))


# speedup vs baseline: 2.7605x; 2.7605x over previous
import functools

import jax
import jax.numpy as jnp
from jax import lax
from jax.experimental import pallas as pl
from jax.experimental.pallas import tpu as pltpu
from jax.experimental.pallas import tpu_sc as plsc

N = 10000
E = 320000
D_IN = 128
D_HID = 256
D_OUT = 128

NC = 2
NS = 16
CH = 128
SUP = 8
NPAD = 10240
ER = 2560
E_PAD = ER * CH

_mesh = plsc.VectorSubcoreMesh(core_axis_name="c", subcore_axis_name="s")


NSUP = ER // SUP


def _degree_body(dst2_hbm, out_hbm, didx_v, ones_v, zb_v, ssem, acc_sh):
  c = lax.axis_index("c")
  s = lax.axis_index("s")
  w = s * NC + c

  @pl.loop(0, CH // 16)
  def _(i):
    ones_v[pl.ds(i * 16, 16)] = jnp.ones((16,), jnp.float32)

  @pl.loop(0, 640 // 16)
  def _(i):
    zb_v[pl.ds(i * 16, 16)] = jnp.zeros((16,), jnp.float32)

  pltpu.sync_copy(zb_v, acc_sh.at[pl.ds(s * 640, 640)])
  plsc.subcore_barrier()

  cnt = NSUP // (NC * NS)

  @pl.loop(0, cnt)
  def _(j):
    r0 = (w * cnt + j) * SUP
    pltpu.sync_copy(dst2_hbm.at[pl.ds(r0, SUP)], didx_v)
    for k in range(SUP):
      pltpu.async_copy(ones_v, acc_sh.at[didx_v.at[k]], ssem, add=True)
    for k in range(SUP):
      pltpu.make_async_copy(ones_v, acc_sh.at[didx_v.at[k]], ssem).wait()

  plsc.subcore_barrier()
  pltpu.sync_copy(acc_sh.at[pl.ds(s * 640, 640)],
                  out_hbm.at[c].at[pl.ds(s * 640, 640)])


_sc_degree = pl.kernel(
    _degree_body,
    out_type=jax.ShapeDtypeStruct((NC, NPAD), jnp.float32),
    mesh=_mesh,
    scratch_types=[
        pltpu.VMEM((SUP, CH), jnp.int32),
        pltpu.VMEM((CH,), jnp.float32),
        pltpu.VMEM((640,), jnp.float32),
        pltpu.SemaphoreType.DMA,
        pltpu.VMEM_SHARED((NPAD,), jnp.float32),
    ],
)


def _scatter_body(split_edges, src2_hbm, dst2_hbm, tbl_hbm, out_hbm,
                  sidx0_v, didx0_v, sidx1_v, didx1_v,
                  rows0_v, rows1_v,
                  gsem0, gsem1, isem0, isem1, ssem, acc_sh):
  c = lax.axis_index("c")
  s = lax.axis_index("s")
  rows = (rows0_v, rows1_v)
  gsems = (gsem0, gsem1)
  sidx = (sidx0_v, sidx1_v)
  didx = (didx0_v, didx1_v)
  isems = (isem0, isem1)

  if split_edges:
    w = s * NC + c
    nw = NC * NS
  else:
    w = s
    nw = NS
  cnt = NSUP // nw

  def row0_of(j):
    return (w * cnt + j) * SUP

  def start_idx(j, ib):
    r0 = row0_of(j)
    pltpu.async_copy(src2_hbm.at[pl.ds(r0, SUP)], sidx[ib], isems[ib])
    pltpu.async_copy(dst2_hbm.at[pl.ds(r0, SUP)], didx[ib], isems[ib])

  def wait_idx(j, ib):
    r0 = row0_of(j)
    pltpu.make_async_copy(src2_hbm.at[pl.ds(r0, SUP)], sidx[ib],
                          isems[ib]).wait()
    pltpu.make_async_copy(dst2_hbm.at[pl.ds(r0, SUP)], didx[ib],
                          isems[ib]).wait()

  start_idx(0, 0)

  @pl.loop(0, CH)
  def _(r):
    @pl.loop(0, 8)
    def _(j):
      rows0_v[r, pl.ds(j * 16, 16)] = jnp.zeros((16,), jnp.float32)

  for i in range(5):
    pltpu.async_copy(rows0_v, acc_sh.at[pl.ds(s * 640 + i * 128, 128)], ssem)
  for i in range(5):
    pltpu.make_async_copy(rows0_v, acc_sh.at[pl.ds(s * 640 + i * 128, 128)],
                          ssem).wait()

  plsc.subcore_barrier()

  tbl_c = tbl_hbm if split_edges else tbl_hbm.at[c]

  def start_gather(idx_row, b):
    pltpu.async_copy(tbl_c.at[idx_row], rows[b], gsems[b])

  def wait_gather(idx_row, b):
    pltpu.make_async_copy(tbl_c.at[idx_row], rows[b], gsems[b]).wait()

  def drain_scatter(b, idx_row):
    pltpu.make_async_copy(rows[b], acc_sh.at[idx_row], ssem).wait()

  def process_super(ib):
    si = sidx[ib]
    di = didx[ib]
    start_gather(si.at[0], 0)
    for k in range(SUP):
      b = k & 1
      if k + 1 < SUP:
        if k >= 1:
          drain_scatter(1 - b, di.at[k - 1])
        start_gather(si.at[k + 1], 1 - b)
      wait_gather(si.at[k], b)
      pltpu.async_copy(rows[b], acc_sh.at[di.at[k]], ssem, add=True)
    drain_scatter(0, di.at[SUP - 2])
    drain_scatter(1, di.at[SUP - 1])

  @pl.loop(0, cnt // 2)
  def _(p):
    j0 = 2 * p
    j1 = j0 + 1
    wait_idx(j0, 0)
    start_idx(j1, 1)
    process_super(0)
    wait_idx(j1, 1)

    @pl.when(j1 + 1 < cnt)
    def _():
      start_idx(j1 + 1, 0)

    process_super(1)

  plsc.subcore_barrier()

  @pl.when(s < NS - 1)
  def _():
    pltpu.sync_copy(acc_sh.at[pl.ds(s * 640, 640)],
                    out_hbm.at[c].at[pl.ds(s * 640, 640)])

  @pl.when(s == NS - 1)
  def _():
    pltpu.sync_copy(acc_sh.at[pl.ds(s * 640, 400)],
                    out_hbm.at[c].at[pl.ds(s * 640, 400)])


def _make_scatter(split_edges):
  return pl.kernel(
      functools.partial(_scatter_body, split_edges),
      out_type=jax.ShapeDtypeStruct((NC, N, 128), jnp.float32),
      mesh=_mesh,
      scratch_types=[
          pltpu.VMEM((SUP, CH), jnp.int32),
          pltpu.VMEM((SUP, CH), jnp.int32),
          pltpu.VMEM((SUP, CH), jnp.int32),
          pltpu.VMEM((SUP, CH), jnp.int32),
          pltpu.VMEM((CH, 128), jnp.float32),
          pltpu.VMEM((CH, 128), jnp.float32),
          pltpu.SemaphoreType.DMA,
          pltpu.SemaphoreType.DMA,
          pltpu.SemaphoreType.DMA,
          pltpu.SemaphoreType.DMA,
          pltpu.SemaphoreType.DMA,
          pltpu.VMEM_SHARED((NPAD, 128), jnp.float32),
      ],
  )


_sc_scatter1 = _make_scatter(False)
_sc_scatter2 = _make_scatter(True)


R = 1000


def _mm1_body(x_ref, w_ref, o_ref):
  o_ref[...] = jnp.dot(x_ref[...], w_ref[...],
                       preferred_element_type=jnp.float32)


def _tc_matmul1(x, W1):
  return pl.pallas_call(
      _mm1_body,
      grid=(N // R,),
      in_specs=[
          pl.BlockSpec((R, D_IN), lambda i: (i, 0)),
          pl.BlockSpec((D_IN, D_HID), lambda i: (0, 0)),
      ],
      out_specs=pl.BlockSpec((R, D_HID), lambda i: (i, 0)),
      out_shape=jax.ShapeDtypeStruct((N, D_HID), jnp.float32),
  )(x, W1)


def _scale_body(degp_ref, xw_ref, dinv_ref, g_ref):
  deg = degp_ref[0] + degp_ref[1] + 1.0
  dinv = lax.rsqrt(deg)
  g = xw_ref[...] * dinv
  dinv_ref[...] = dinv
  g_ref[0] = g[:, :128]
  g_ref[1] = g[:, 128:]


def _tc_scale(degp, xw):
  return pl.pallas_call(
      _scale_body,
      grid=(N // R,),
      in_specs=[
          pl.BlockSpec((NC, R, 1), lambda i: (0, i, 0)),
          pl.BlockSpec((R, D_HID), lambda i: (i, 0)),
      ],
      out_specs=[
          pl.BlockSpec((R, 1), lambda i: (i, 0)),
          pl.BlockSpec((NC, R, 128), lambda i: (0, i, 0)),
      ],
      out_shape=[
          jax.ShapeDtypeStruct((N, 1), jnp.float32),
          jax.ShapeDtypeStruct((NC, N, 128), jnp.float32),
      ],
  )(degp, xw)


def _mid_body(a_ref, g1_ref, dinv_ref, b1_ref, w2_ref, g2_ref):
  dinv = dinv_ref[...]
  h_lo = jnp.maximum(dinv * (a_ref[0] + g1_ref[0]) + b1_ref[:, :128], 0.0)
  h_hi = jnp.maximum(dinv * (a_ref[1] + g1_ref[1]) + b1_ref[:, 128:], 0.0)
  xw2 = (jnp.dot(h_lo, w2_ref[:128], preferred_element_type=jnp.float32) +
         jnp.dot(h_hi, w2_ref[128:], preferred_element_type=jnp.float32))
  g2_ref[...] = dinv * xw2


def _tc_mid(a1, g1, dinv, b1, W2):
  return pl.pallas_call(
      _mid_body,
      grid=(N // R,),
      in_specs=[
          pl.BlockSpec((NC, R, 128), lambda i: (0, i, 0)),
          pl.BlockSpec((NC, R, 128), lambda i: (0, i, 0)),
          pl.BlockSpec((R, 1), lambda i: (i, 0)),
          pl.BlockSpec((1, D_HID), lambda i: (0, 0)),
          pl.BlockSpec((D_HID, D_OUT), lambda i: (0, 0)),
      ],
      out_specs=pl.BlockSpec((R, 128), lambda i: (i, 0)),
      out_shape=jax.ShapeDtypeStruct((N, 128), jnp.float32),
  )(a1, g1, dinv, b1, W2)


def _post_body(a2_ref, g2_ref, dinv_ref, b2_ref, o_ref):
  o_ref[...] = (dinv_ref[...] * (a2_ref[0] + a2_ref[1] + g2_ref[...])
                + b2_ref[...])


def _tc_post(a2, g2, dinv, b2):
  return pl.pallas_call(
      _post_body,
      grid=(N // R,),
      in_specs=[
          pl.BlockSpec((NC, R, 128), lambda i: (0, i, 0)),
          pl.BlockSpec((R, 128), lambda i: (i, 0)),
          pl.BlockSpec((R, 1), lambda i: (i, 0)),
          pl.BlockSpec((1, D_OUT), lambda i: (0, 0)),
      ],
      out_specs=pl.BlockSpec((R, D_OUT), lambda i: (i, 0)),
      out_shape=jax.ShapeDtypeStruct((N, D_OUT), jnp.float32),
  )(a2, g2, dinv, b2)


def kernel(x, edge_index, W1, b1, W2, b2):
  npad = E_PAD - E
  pad_iota = lax.iota(jnp.int32, npad)
  src2 = jnp.concatenate(
      [edge_index[0], pad_iota % N]).reshape(ER, CH)
  dst2 = jnp.concatenate(
      [edge_index[1], N + pad_iota % (NPAD - N)]).reshape(ER, CH)
  xw = _tc_matmul1(x, W1)
  degp = _sc_degree(dst2)
  dinv, g1 = _tc_scale(degp.reshape(NC, NPAD, 1), xw)
  a1 = _sc_scatter1(src2, dst2, g1)
  g2 = _tc_mid(a1, g1, dinv, b1.reshape(1, D_HID), W2)
  a2 = _sc_scatter2(src2, dst2, g2)
  return _tc_post(a2, g2, dinv, b2.reshape(1, D_OUT))


# speedup vs baseline: 3.0454x; 1.1032x over previous
import functools

import jax
import jax.numpy as jnp
from jax import lax
from jax.experimental import pallas as pl
from jax.experimental.pallas import tpu as pltpu
from jax.experimental.pallas import tpu_sc as plsc

N = 10000
E = 320000
D_IN = 128
D_HID = 256
D_OUT = 128

NC = 2
NS = 16
CH = 128
SUP = 16
NPAD = 10240
ER = 2560
E_PAD = ER * CH

_mesh = plsc.VectorSubcoreMesh(core_axis_name="c", subcore_axis_name="s")


NSUP = ER // SUP


def _degree_body(dst2_hbm, out_hbm, didx_v, ones_v, zb_v, ssem, acc_sh):
  c = lax.axis_index("c")
  s = lax.axis_index("s")
  w = s * NC + c

  @pl.loop(0, CH // 16)
  def _(i):
    ones_v[pl.ds(i * 16, 16)] = jnp.ones((16,), jnp.float32)

  @pl.loop(0, 640 // 16)
  def _(i):
    zb_v[pl.ds(i * 16, 16)] = jnp.zeros((16,), jnp.float32)

  pltpu.sync_copy(zb_v, acc_sh.at[pl.ds(s * 640, 640)])
  plsc.subcore_barrier()

  cnt = NSUP // (NC * NS)

  @pl.loop(0, cnt)
  def _(j):
    r0 = (w * cnt + j) * SUP
    pltpu.sync_copy(dst2_hbm.at[pl.ds(r0, SUP)], didx_v)
    for k in range(SUP):
      pltpu.async_copy(ones_v, acc_sh.at[didx_v.at[k]], ssem, add=True)
    for k in range(SUP):
      pltpu.make_async_copy(ones_v, acc_sh.at[didx_v.at[k]], ssem).wait()

  plsc.subcore_barrier()
  pltpu.sync_copy(acc_sh.at[pl.ds(s * 640, 640)],
                  out_hbm.at[c].at[pl.ds(s * 640, 640)])


_sc_degree = pl.kernel(
    _degree_body,
    out_type=jax.ShapeDtypeStruct((NC, NPAD), jnp.float32),
    mesh=_mesh,
    scratch_types=[
        pltpu.VMEM((SUP, CH), jnp.int32),
        pltpu.VMEM((CH,), jnp.float32),
        pltpu.VMEM((640,), jnp.float32),
        pltpu.SemaphoreType.DMA,
        pltpu.VMEM_SHARED((NPAD,), jnp.float32),
    ],
)


def _scatter_body(split_edges, src2_hbm, dst2_hbm, tbl_hbm, out_hbm,
                  sidx0_v, didx0_v, sidx1_v, didx1_v,
                  rows0_v, rows1_v,
                  gsem0, gsem1, isem0, isem1, ssem, acc_sh):
  c = lax.axis_index("c")
  s = lax.axis_index("s")
  rows = (rows0_v, rows1_v)
  gsems = (gsem0, gsem1)
  sidx = (sidx0_v, sidx1_v)
  didx = (didx0_v, didx1_v)
  isems = (isem0, isem1)

  if split_edges:
    w = s * NC + c
    nw = NC * NS
  else:
    w = s
    nw = NS
  cnt = NSUP // nw

  def row0_of(j):
    return (w * cnt + j) * SUP

  def start_idx(j, ib):
    r0 = row0_of(j)
    pltpu.async_copy(src2_hbm.at[pl.ds(r0, SUP)], sidx[ib], isems[ib])
    pltpu.async_copy(dst2_hbm.at[pl.ds(r0, SUP)], didx[ib], isems[ib])

  def wait_idx(j, ib):
    r0 = row0_of(j)
    pltpu.make_async_copy(src2_hbm.at[pl.ds(r0, SUP)], sidx[ib],
                          isems[ib]).wait()
    pltpu.make_async_copy(dst2_hbm.at[pl.ds(r0, SUP)], didx[ib],
                          isems[ib]).wait()

  start_idx(0, 0)

  @pl.loop(0, CH)
  def _(r):
    @pl.loop(0, 8)
    def _(j):
      rows0_v[r, pl.ds(j * 16, 16)] = jnp.zeros((16,), jnp.float32)

  for i in range(5):
    pltpu.async_copy(rows0_v, acc_sh.at[pl.ds(s * 640 + i * 128, 128)], ssem)
  for i in range(5):
    pltpu.make_async_copy(rows0_v, acc_sh.at[pl.ds(s * 640 + i * 128, 128)],
                          ssem).wait()

  plsc.subcore_barrier()

  tbl_c = tbl_hbm if split_edges else tbl_hbm.at[c]

  def start_gather(idx_row, b):
    pltpu.async_copy(tbl_c.at[idx_row], rows[b], gsems[b])

  def wait_gather(idx_row, b):
    pltpu.make_async_copy(tbl_c.at[idx_row], rows[b], gsems[b]).wait()

  def drain_scatter(b, idx_row):
    pltpu.make_async_copy(rows[b], acc_sh.at[idx_row], ssem).wait()

  def process_super(ib):
    si = sidx[ib]
    di = didx[ib]
    start_gather(si.at[0], 0)
    for k in range(SUP):
      b = k & 1
      if k + 1 < SUP:
        if k >= 1:
          drain_scatter(1 - b, di.at[k - 1])
        start_gather(si.at[k + 1], 1 - b)
      wait_gather(si.at[k], b)
      pltpu.async_copy(rows[b], acc_sh.at[di.at[k]], ssem, add=True)
    drain_scatter(0, di.at[SUP - 2])
    drain_scatter(1, di.at[SUP - 1])

  @pl.loop(0, cnt // 2)
  def _(p):
    j0 = 2 * p
    j1 = j0 + 1
    wait_idx(j0, 0)
    start_idx(j1, 1)
    process_super(0)
    wait_idx(j1, 1)

    @pl.when(j1 + 1 < cnt)
    def _():
      start_idx(j1 + 1, 0)

    process_super(1)

  plsc.subcore_barrier()

  @pl.when(s < NS - 1)
  def _():
    pltpu.sync_copy(acc_sh.at[pl.ds(s * 640, 640)],
                    out_hbm.at[c].at[pl.ds(s * 640, 640)])

  @pl.when(s == NS - 1)
  def _():
    pltpu.sync_copy(acc_sh.at[pl.ds(s * 640, 400)],
                    out_hbm.at[c].at[pl.ds(s * 640, 400)])


def _make_scatter(split_edges):
  return pl.kernel(
      functools.partial(_scatter_body, split_edges),
      out_type=jax.ShapeDtypeStruct((NC, N, 128), jnp.float32),
      mesh=_mesh,
      scratch_types=[
          pltpu.VMEM((SUP, CH), jnp.int32),
          pltpu.VMEM((SUP, CH), jnp.int32),
          pltpu.VMEM((SUP, CH), jnp.int32),
          pltpu.VMEM((SUP, CH), jnp.int32),
          pltpu.VMEM((CH, 128), jnp.float32),
          pltpu.VMEM((CH, 128), jnp.float32),
          pltpu.SemaphoreType.DMA,
          pltpu.SemaphoreType.DMA,
          pltpu.SemaphoreType.DMA,
          pltpu.SemaphoreType.DMA,
          pltpu.SemaphoreType.DMA,
          pltpu.VMEM_SHARED((NPAD, 128), jnp.float32),
      ],
  )


_sc_scatter1 = _make_scatter(False)
_sc_scatter2 = _make_scatter(True)


R = 1000


def _mm1_body(x_ref, w_ref, o_ref):
  o_ref[...] = jnp.dot(x_ref[...], w_ref[...],
                       preferred_element_type=jnp.float32)


def _tc_matmul1(x, W1):
  return pl.pallas_call(
      _mm1_body,
      grid=(N // R,),
      in_specs=[
          pl.BlockSpec((R, D_IN), lambda i: (i, 0)),
          pl.BlockSpec((D_IN, D_HID), lambda i: (0, 0)),
      ],
      out_specs=pl.BlockSpec((R, D_HID), lambda i: (i, 0)),
      out_shape=jax.ShapeDtypeStruct((N, D_HID), jnp.float32),
  )(x, W1)


def _scale_body(degp_ref, xw_ref, dinv_ref, g_ref):
  deg = degp_ref[0] + degp_ref[1] + 1.0
  dinv = lax.rsqrt(deg)
  g = xw_ref[...] * dinv
  dinv_ref[...] = dinv
  g_ref[0] = g[:, :128]
  g_ref[1] = g[:, 128:]


def _tc_scale(degp, xw):
  return pl.pallas_call(
      _scale_body,
      grid=(N // R,),
      in_specs=[
          pl.BlockSpec((NC, R, 1), lambda i: (0, i, 0)),
          pl.BlockSpec((R, D_HID), lambda i: (i, 0)),
      ],
      out_specs=[
          pl.BlockSpec((R, 1), lambda i: (i, 0)),
          pl.BlockSpec((NC, R, 128), lambda i: (0, i, 0)),
      ],
      out_shape=[
          jax.ShapeDtypeStruct((N, 1), jnp.float32),
          jax.ShapeDtypeStruct((NC, N, 128), jnp.float32),
      ],
  )(degp, xw)


def _mid_body(a_ref, g1_ref, dinv_ref, b1_ref, w2_ref, g2_ref):
  dinv = dinv_ref[...]
  h_lo = jnp.maximum(dinv * (a_ref[0] + g1_ref[0]) + b1_ref[:, :128], 0.0)
  h_hi = jnp.maximum(dinv * (a_ref[1] + g1_ref[1]) + b1_ref[:, 128:], 0.0)
  xw2 = (jnp.dot(h_lo, w2_ref[:128], preferred_element_type=jnp.float32) +
         jnp.dot(h_hi, w2_ref[128:], preferred_element_type=jnp.float32))
  g2_ref[...] = dinv * xw2


def _tc_mid(a1, g1, dinv, b1, W2):
  return pl.pallas_call(
      _mid_body,
      grid=(N // R,),
      in_specs=[
          pl.BlockSpec((NC, R, 128), lambda i: (0, i, 0)),
          pl.BlockSpec((NC, R, 128), lambda i: (0, i, 0)),
          pl.BlockSpec((R, 1), lambda i: (i, 0)),
          pl.BlockSpec((1, D_HID), lambda i: (0, 0)),
          pl.BlockSpec((D_HID, D_OUT), lambda i: (0, 0)),
      ],
      out_specs=pl.BlockSpec((R, 128), lambda i: (i, 0)),
      out_shape=jax.ShapeDtypeStruct((N, 128), jnp.float32),
  )(a1, g1, dinv, b1, W2)


def _post_body(a2_ref, g2_ref, dinv_ref, b2_ref, o_ref):
  o_ref[...] = (dinv_ref[...] * (a2_ref[0] + a2_ref[1] + g2_ref[...])
                + b2_ref[...])


def _tc_post(a2, g2, dinv, b2):
  return pl.pallas_call(
      _post_body,
      grid=(N // R,),
      in_specs=[
          pl.BlockSpec((NC, R, 128), lambda i: (0, i, 0)),
          pl.BlockSpec((R, 128), lambda i: (i, 0)),
          pl.BlockSpec((R, 1), lambda i: (i, 0)),
          pl.BlockSpec((1, D_OUT), lambda i: (0, 0)),
      ],
      out_specs=pl.BlockSpec((R, D_OUT), lambda i: (i, 0)),
      out_shape=jax.ShapeDtypeStruct((N, D_OUT), jnp.float32),
  )(a2, g2, dinv, b2)


def kernel(x, edge_index, W1, b1, W2, b2):
  npad = E_PAD - E
  pad_iota = lax.iota(jnp.int32, npad)
  src2 = jnp.concatenate(
      [edge_index[0], pad_iota % N]).reshape(ER, CH)
  dst2 = jnp.concatenate(
      [edge_index[1], N + pad_iota % (NPAD - N)]).reshape(ER, CH)
  xw = _tc_matmul1(x, W1)
  degp = _sc_degree(dst2)
  dinv, g1 = _tc_scale(degp.reshape(NC, NPAD, 1), xw)
  a1 = _sc_scatter1(src2, dst2, g1)
  g2 = _tc_mid(a1, g1, dinv, b1.reshape(1, D_HID), W2)
  a2 = _sc_scatter2(src2, dst2, g2)
  return _tc_post(a2, g2, dinv, b2.reshape(1, D_OUT))
